# baseline (speedup 1.0000x reference)
import numpy as np
import jax
import jax.numpy as jnp
from jax.experimental import pallas as pl

DIM = 256
H = 8
A = 64
DH = 32
BETA = 0.8
NJ = 24
N_SKEL = 17
N_CONVS = 4


def _skeleton_edges():
    bones = [(0, 1), (1, 2), (2, 3), (0, 4), (4, 5), (5, 6), (0, 7), (7, 8),
             (8, 9), (9, 10), (8, 11), (11, 12), (12, 13), (8, 14), (14, 15), (15, 16)]
    src, dst = [], []
    for a, b in bones:
        src += [a, b]
        dst += [b, a]
    for i in range(N_SKEL):
        src.append(i)
        dst.append(i)
    return src, dst


def _cayley_edges():
    elems = []
    for a in range(3):
        for b in range(3):
            for c in range(3):
                for d in range(3):
                    if (a * d - b * c) % 3 == 1:
                        elems.append((a, b, c, d))
    idx = {e: i for i, e in enumerate(elems)}
    gens = [((1, 1), (0, 1)), ((1, 2), (0, 1)), ((1, 0), (1, 1)), ((1, 0), (2, 1))]
    src, dst = [], []
    for (a, b, c, d) in elems:
        for ((p, q), (r, s)) in gens:
            ne = ((p * a + q * c) % 3, (p * b + q * d) % 3,
                  (r * a + s * c) % 3, (r * b + s * d) % 3)
            src.append(idx[(a, b, c, d)])
            dst.append(idx[ne])
    for i in range(NJ):
        src.append(i)
        dst.append(i)
    return src, dst


def _mask(src, dst):
    m = np.full((NJ, NJ), -1e30, np.float32)
    m[np.array(src), np.array(dst)] = 0.0
    return m

_MADD = np.stack([_mask(*_skeleton_edges()), _mask(*_cayley_edges())] * 2)


def _layernorm(x, g, b, eps=1e-5):
    mu = jnp.mean(x, axis=-1, keepdims=True)
    var = jnp.mean(jnp.square(x - mu), axis=-1, keepdims=True)
    return (x - mu) * jax.lax.rsqrt(var + eps) * g + b


def _body(x_ref, w_ref, a_ref, lng_ref, lnb_ref, pw_ref, pb_ref,
          plng_ref, plnb_ref, madd_ref, o_ref):
    X = x_ref[...]
    R = X.shape[0]
    state = X
    acc = jnp.dot(X.reshape(R * NJ, DIM), pw_ref[0],
                  preferred_element_type=jnp.float32)
    for i in range(N_CONVS):
        qkv = jnp.dot(state.reshape(R * NJ, DIM), w_ref[i],
                      preferred_element_type=jnp.float32)
        a_vec = a_ref[i, 0, :]
        madd = madd_ref[i]
        cols = []
        for h in range(H):
            b0 = h * 3 * A
            q = qkv[:, b0:b0 + A].reshape(R, NJ, A)
            k = qkv[:, b0 + A:b0 + 2 * A].reshape(R, NJ, A)
            v1 = qkv[:, b0 + 2 * A:b0 + 2 * A + DH].reshape(R, NJ, DH)
            v2 = qkv[:, b0 + 2 * A + DH:b0 + 3 * A].reshape(R, NJ, DH)
            sp = jax.nn.softplus(q[:, :, None, :] + k[:, None, :, :])
            z = jnp.sum(sp * a_vec, axis=-1) + madd
            zmax = jnp.max(z, axis=(1, 2), keepdims=True)
            p = jnp.exp(z - zmax)
            sig = jnp.sum(p, axis=2, keepdims=True) + 1e-10
            d = p / sig
            m = jax.lax.dot_general(d, v2, (((2,), (1,)), ((0,), (0,))),
                                    preferred_element_type=jnp.float32)
            cols.append((1.0 - BETA) * v1 + BETA * m)
        V = jnp.concatenate(cols, axis=-1)
        state = _layernorm(state + V, lng_ref[i], lnb_ref[i])
        acc = acc + jnp.dot(state.reshape(R * NJ, DIM), pw_ref[i + 1],
                            preferred_element_type=jnp.float32)
    out = acc + pb_ref[0]
    out = _layernorm(out, plng_ref[0], plnb_ref[0])
    o_ref[...] = out.reshape(R, NJ, DIM)


def kernel(x, w_qkv_all, a_all, ln_g, ln_b, proj_w, proj_b, pln_g, pln_b):
    B, T, J, C = x.shape
    N = B * T
    R = 24
    xp = jnp.concatenate(
        [x, jnp.zeros((B, T, NJ - J, C), x.dtype)], axis=2).reshape(N, NJ, C)
    wT = jnp.transpose(w_qkv_all, (0, 2, 1))
    pw = proj_w.T.reshape(N_CONVS + 1, C, C)
    madd = jnp.asarray(_MADD)

    out = pl.pallas_call(
        _body,
        grid=(N // R,),
        in_specs=[
            pl.BlockSpec((R, NJ, C), lambda i: (i, 0, 0)),
            pl.BlockSpec(wT.shape, lambda i: (0, 0, 0)),
            pl.BlockSpec(a_all.shape, lambda i: (0, 0, 0)),
            pl.BlockSpec(ln_g.shape, lambda i: (0, 0)),
            pl.BlockSpec(ln_b.shape, lambda i: (0, 0)),
            pl.BlockSpec(pw.shape, lambda i: (0, 0, 0)),
            pl.BlockSpec((1, C), lambda i: (0, 0)),
            pl.BlockSpec((1, C), lambda i: (0, 0)),
            pl.BlockSpec((1, C), lambda i: (0, 0)),
            pl.BlockSpec(madd.shape, lambda i: (0, 0, 0)),
        ],
        out_specs=pl.BlockSpec((R, NJ, C), lambda i: (i, 0, 0)),
        out_shape=jax.ShapeDtypeStruct((N, NJ, C), jnp.float32),
    )(xp, wT, a_all, ln_g, ln_b, pw, proj_b.reshape(1, C),
      pln_g.reshape(1, C), pln_b.reshape(1, C), madd)

    return out.reshape(B, T, NJ, C)[:, :, :J, :]
